# Initial kernel scaffold; baseline (speedup 1.0000x reference)
import jax
import jax.numpy as jnp
from jax.experimental import pallas as pl


def kernel(states_batch, rule_embedding):
    raise NotImplementedError("write your pallas kernel here")


# speedup vs baseline: 2.8442x; 2.8442x over previous
import functools

import jax
import jax.numpy as jnp
from jax import lax
from jax.experimental import pallas as pl
from jax.experimental.pallas import tpu as pltpu
from jax.experimental.pallas import tpu_sc as plsc

N_RULES = 1000
D_MODEL = 512
BATCH = 1024
SEQ = 200

NW = 32
ROWS = SEQ * BATCH
ROWS_PER_W = ROWS // NW
CHUNK = 64
CHUNKS_PER_W = ROWS_PER_W // CHUNK


def _make_sc_gather():
    mesh = plsc.VectorSubcoreMesh(core_axis_name="c", subcore_axis_name="s")

    @functools.partial(
        pl.kernel,
        mesh=mesh,
        out_type=jax.ShapeDtypeStruct((ROWS, D_MODEL), jnp.float32),
        scratch_types=[
            pltpu.VMEM((CHUNKS_PER_W, CHUNK), jnp.int32),
            pltpu.VMEM((CHUNK, D_MODEL), jnp.float32),
            pltpu.SemaphoreType.DMA,
        ],
    )
    def k(table_hbm, idx_hbm, out_hbm, idx_v, rows_v, sem):
        wid = lax.axis_index("s") * 2 + lax.axis_index("c")
        base = wid * ROWS_PER_W
        pltpu.sync_copy(idx_hbm.at[wid], idx_v)

        def step(j, carry):
            pltpu.async_copy(table_hbm.at[idx_v.at[j]], rows_v, sem).wait()
            pltpu.sync_copy(rows_v, out_hbm.at[pl.ds(base + j * CHUNK, CHUNK)])
            return carry

        lax.fori_loop(0, CHUNKS_PER_W, step, 0)

    return k


_sc_gather = _make_sc_gather()


def kernel(states_batch, rule_embedding):
    idx_t = states_batch.T.reshape(NW, CHUNKS_PER_W, CHUNK)
    out = _sc_gather(rule_embedding, idx_t)
    return out.reshape(SEQ, BATCH, D_MODEL)


# speedup vs baseline: 3.2515x; 1.1432x over previous
import functools

import jax
import jax.numpy as jnp
from jax import lax
from jax.experimental import pallas as pl
from jax.experimental.pallas import tpu as pltpu
from jax.experimental.pallas import tpu_sc as plsc

N_RULES = 1000
D_MODEL = 512
BATCH = 1024
SEQ = 200

NW = 32
ROWS = SEQ * BATCH
ROWS_PER_W = ROWS // NW
CHUNK = 64
CHUNKS_PER_W = ROWS_PER_W // CHUNK


def _make_sc_gather():
    mesh = plsc.VectorSubcoreMesh(core_axis_name="c", subcore_axis_name="s")

    @functools.partial(
        pl.kernel,
        mesh=mesh,
        out_type=jax.ShapeDtypeStruct((ROWS, D_MODEL), jnp.float32),
        scratch_types=[
            pltpu.VMEM((CHUNKS_PER_W, CHUNK), jnp.int32),
            pltpu.VMEM((2, CHUNK, D_MODEL), jnp.float32),
            pltpu.SemaphoreType.DMA,
            pltpu.SemaphoreType.DMA,
        ],
    )
    def k(table_hbm, idx_hbm, out_hbm, idx_v, rows_v, sem0, sem1):
        wid = lax.axis_index("s") * 2 + lax.axis_index("c")
        base = wid * ROWS_PER_W
        pltpu.sync_copy(idx_hbm.at[wid], idx_v)
        sems = (sem0, sem1)

        pltpu.async_copy(table_hbm.at[idx_v.at[0]], rows_v.at[0], sem0)

        def step(j2, carry):
            for s in (0, 1):
                j = j2 * 2 + s
                nxt = j + 1

                @pl.when(nxt < CHUNKS_PER_W)
                def _():
                    pltpu.async_copy(
                        table_hbm.at[idx_v.at[nxt]], rows_v.at[1 - s], sems[1 - s]
                    )

                pltpu.make_async_copy(
                    table_hbm.at[idx_v.at[j]], rows_v.at[s], sems[s]
                ).wait()
                pltpu.sync_copy(
                    rows_v.at[s], out_hbm.at[pl.ds(base + j * CHUNK, CHUNK)]
                )
            return carry

        lax.fori_loop(0, CHUNKS_PER_W // 2, step, 0)

    return k


_sc_gather = _make_sc_gather()


def kernel(states_batch, rule_embedding):
    idx_t = states_batch.T.reshape(NW, CHUNKS_PER_W, CHUNK)
    out = _sc_gather(rule_embedding, idx_t)
    return out.reshape(SEQ, BATCH, D_MODEL)


# speedup vs baseline: 3.3060x; 1.0168x over previous
import functools

import jax
import jax.numpy as jnp
from jax import lax
from jax.experimental import pallas as pl
from jax.experimental.pallas import tpu as pltpu
from jax.experimental.pallas import tpu_sc as plsc

N_RULES = 1000
D_MODEL = 512
BATCH = 1024
SEQ = 200

NW = 32
ROWS = SEQ * BATCH
ROWS_PER_W = ROWS // NW
CHUNK = 40
CHUNKS_PER_W = ROWS_PER_W // CHUNK


def _make_sc_gather():
    mesh = plsc.VectorSubcoreMesh(core_axis_name="c", subcore_axis_name="s")

    @functools.partial(
        pl.kernel,
        mesh=mesh,
        out_type=jax.ShapeDtypeStruct((ROWS, D_MODEL), jnp.float32),
        scratch_types=[
            pltpu.VMEM((CHUNKS_PER_W, CHUNK), jnp.int32),
            pltpu.VMEM((2, CHUNK, D_MODEL), jnp.float32),
            pltpu.VMEM_SHARED((16, 2, CHUNK, D_MODEL), jnp.float32),
            pltpu.SemaphoreType.DMA,
            pltpu.SemaphoreType.DMA,
            pltpu.SemaphoreType.DMA,
            pltpu.SemaphoreType.DMA,
        ],
    )
    def k(table_hbm, idx_hbm, out_hbm, idx_v, rows_v, stage_sh,
          sem0, sem1, wsem0, wsem1):
        sid = lax.axis_index("s")
        wid = sid * 2 + lax.axis_index("c")
        base = wid * ROWS_PER_W
        slots = stage_sh.at[sid]

        pltpu.sync_copy(idx_hbm.at[wid], idx_v)
        sems = (sem0, sem1)
        wsems = (wsem0, wsem1)

        pltpu.async_copy(table_hbm.at[idx_v.at[0]], rows_v.at[0], sem0)

        def step(j2, carry):
            for s in (0, 1):
                j = j2 * 2 + s
                nxt = j + 1

                @pl.when(nxt < CHUNKS_PER_W)
                def _():
                    pltpu.async_copy(
                        table_hbm.at[idx_v.at[nxt]], rows_v.at[1 - s], sems[1 - s]
                    )

                pltpu.make_async_copy(
                    table_hbm.at[idx_v.at[j]], rows_v.at[s], sems[s]
                ).wait()

                @pl.when(j >= 2)
                def _():
                    pltpu.make_async_copy(
                        slots.at[s], out_hbm.at[pl.ds(base, CHUNK)], wsems[s]
                    ).wait()

                pltpu.sync_copy(rows_v.at[s], slots.at[s])
                pltpu.async_copy(
                    slots.at[s], out_hbm.at[pl.ds(base + j * CHUNK, CHUNK)],
                    wsems[s],
                )
            return carry

        lax.fori_loop(0, CHUNKS_PER_W // 2, step, 0)

        for s in (0, 1):
            pltpu.make_async_copy(
                slots.at[s], out_hbm.at[pl.ds(base, CHUNK)], wsems[s]
            ).wait()

    return k


_sc_gather = _make_sc_gather()


def kernel(states_batch, rule_embedding):
    idx_t = states_batch.T.reshape(NW, CHUNKS_PER_W, CHUNK)
    out = _sc_gather(rule_embedding, idx_t)
    return out.reshape(SEQ, BATCH, D_MODEL)
